# Initial kernel scaffold; baseline (speedup 1.0000x reference)
import jax
import jax.numpy as jnp
from jax.experimental import pallas as pl


def kernel(x, edge_index, W, b):
    raise NotImplementedError("write your pallas kernel here")


# speedup vs baseline: 76.2008x; 76.2008x over previous
import functools

import jax
import jax.numpy as jnp
from jax import lax
from jax.experimental import pallas as pl
from jax.experimental.pallas import tpu as pltpu
from jax.experimental.pallas import tpu_sc as plsc

_N = 100000
_E = 3200000
_OUT = 4
_LANE = 128
_NC = 2
_NS = 16
_ROWS = 25600
_RPW = _ROWS // (_NC * _NS)
_RC = 40
_NCHUNK = _RPW // _RC
_PADN = 7680
_PADR = 10
_DSL = 6248
_DSL_LAST = _N - (_NS - 1) * _DSL

_mesh = plsc.VectorSubcoreMesh(core_axis_name="core", subcore_axis_name="subcore")
_sc_params = pltpu.CompilerParams(use_tc_tiling_on_sc=False)


def _per_tile_slice(s, fn):
    @pl.when(s < _NS - 1)
    def _():
        fn(s * _DSL, _DSL)

    @pl.when(s == _NS - 1)
    def _():
        fn((_NS - 1) * _DSL, _DSL_LAST)


@functools.partial(
    pl.kernel,
    out_type=(
        jax.ShapeDtypeStruct((_N, 1), jnp.float32),
        jax.ShapeDtypeStruct((_N, 1), jnp.float32),
    ),
    mesh=_mesh,
    scratch_types=[
        pltpu.VMEM((_RC, _LANE), jnp.int32),
        pltpu.VMEM((_LANE, 1), jnp.float32),
        pltpu.VMEM_SHARED((_N, 1), jnp.float32),
    ],
    compiler_params=_sc_params,
)
def _deg_kernel(edges, zeros, ones, deg0, deg1, idxv, onesv, degsp):
    c = lax.axis_index("core")
    s = lax.axis_index("subcore")
    w = c * _NS + s

    pltpu.sync_copy(ones, onesv)

    def _zero(start, size):
        pltpu.sync_copy(zeros.at[pl.ds(start, size)],
                        degsp.at[pl.ds(start, size)])

    _per_tile_slice(s, _zero)
    plsc.subcore_barrier()

    @pl.loop(0, _NCHUNK)
    def _chunk(ci):
        row0 = w * _RPW + ci * _RC
        pltpu.sync_copy(edges.at[1, pl.ds(row0, _RC)], idxv)

        @pl.loop(0, _RC)
        def _row(j):
            pltpu.sync_copy(onesv, degsp.at[idxv.at[j]], add=True)

    plsc.subcore_barrier()

    def _read(start, size):
        @pl.when(c == 0)
        def _():
            pltpu.sync_copy(degsp.at[pl.ds(start, size)],
                            deg0.at[pl.ds(start, size)])

        @pl.when(c == 1)
        def _():
            pltpu.sync_copy(degsp.at[pl.ds(start, size)],
                            deg1.at[pl.ds(start, size)])

    _per_tile_slice(s, _read)


@functools.partial(
    pl.kernel,
    out_type=(
        jax.ShapeDtypeStruct((_N, _OUT), jnp.float32),
        jax.ShapeDtypeStruct((_N, _OUT), jnp.float32),
    ),
    mesh=_mesh,
    scratch_types=[
        pltpu.VMEM((_RC, _LANE), jnp.int32),
        pltpu.VMEM((_RC, _LANE), jnp.int32),
        pltpu.VMEM((_LANE, _OUT), jnp.float32),
        pltpu.VMEM_SHARED((_N, _OUT), jnp.float32),
        pltpu.VMEM_SHARED((_N, _OUT), jnp.float32),
    ],
    compiler_params=_sc_params,
)
def _agg_kernel(edges, p, acc0, acc1, srcv, dstv, rowbuf, psp, accsp):
    c = lax.axis_index("core")
    s = lax.axis_index("subcore")
    w = c * _NS + s

    def _stage(start, size):
        pltpu.sync_copy(p.at[pl.ds(start, size)], psp.at[pl.ds(start, size)])
        pltpu.sync_copy(p.at[pl.ds(start, size)], accsp.at[pl.ds(start, size)])

    _per_tile_slice(s, _stage)
    plsc.subcore_barrier()

    @pl.loop(0, _NCHUNK)
    def _chunk(ci):
        row0 = w * _RPW + ci * _RC
        pltpu.sync_copy(edges.at[0, pl.ds(row0, _RC)], srcv)
        pltpu.sync_copy(edges.at[1, pl.ds(row0, _RC)], dstv)

        @pl.loop(0, _RC)
        def _row(j):
            pltpu.sync_copy(psp.at[srcv.at[j]], rowbuf)
            pltpu.sync_copy(rowbuf, accsp.at[dstv.at[j]], add=True)

    plsc.subcore_barrier()

    def _read(start, size):
        @pl.when(c == 0)
        def _():
            pltpu.sync_copy(accsp.at[pl.ds(start, size)],
                            acc0.at[pl.ds(start, size)])

        @pl.when(c == 1)
        def _():
            pltpu.sync_copy(accsp.at[pl.ds(start, size)],
                            acc1.at[pl.ds(start, size)])

    _per_tile_slice(s, _read)


_BN = 5000


def _prep_body(d0_ref, d1_ref, x_ref, w_ref, p_ref, dis_ref):
    i = pl.program_id(0)
    gid = lax.broadcasted_iota(jnp.int32, (_BN, 1), 0) + i * _BN
    mask = (gid < _PADN).astype(jnp.float32)
    degt = d0_ref[...] + d1_ref[...] + 1.0 - float(_PADR) * mask
    dis = lax.rsqrt(degt)
    h = jnp.dot(x_ref[...], w_ref[...], preferred_element_type=jnp.float32)
    p_ref[...] = h * dis
    dis_ref[...] = dis


_prep = pl.pallas_call(
    _prep_body,
    grid=(_N // _BN,),
    in_specs=[
        pl.BlockSpec((_BN, 1), lambda i: (i, 0)),
        pl.BlockSpec((_BN, 1), lambda i: (i, 0)),
        pl.BlockSpec((_BN, 16), lambda i: (i, 0)),
        pl.BlockSpec((16, _OUT), lambda i: (0, 0)),
    ],
    out_specs=[
        pl.BlockSpec((_BN, _OUT), lambda i: (i, 0)),
        pl.BlockSpec((_BN, 1), lambda i: (i, 0)),
    ],
    out_shape=[
        jax.ShapeDtypeStruct((_N, _OUT), jnp.float32),
        jax.ShapeDtypeStruct((_N, 1), jnp.float32),
    ],
)


def _fin_body(a0_ref, a1_ref, p_ref, dis_ref, b_ref, o_ref):
    i = pl.program_id(0)
    gid = lax.broadcasted_iota(jnp.int32, (_BN, 1), 0) + i * _BN
    mask = (gid < _PADN).astype(jnp.float32)
    o_ref[...] = dis_ref[...] * (
        a0_ref[...] + a1_ref[...] - p_ref[...] * (1.0 + float(_PADR) * mask)
    ) + b_ref[...]


_fin = pl.pallas_call(
    _fin_body,
    grid=(_N // _BN,),
    in_specs=[
        pl.BlockSpec((_BN, _OUT), lambda i: (i, 0)),
        pl.BlockSpec((_BN, _OUT), lambda i: (i, 0)),
        pl.BlockSpec((_BN, _OUT), lambda i: (i, 0)),
        pl.BlockSpec((_BN, 1), lambda i: (i, 0)),
        pl.BlockSpec((1, _OUT), lambda i: (0, 0)),
    ],
    out_specs=pl.BlockSpec((_BN, _OUT), lambda i: (i, 0)),
    out_shape=jax.ShapeDtypeStruct((_N, _OUT), jnp.float32),
)


@jax.jit
def kernel(x, edge_index, W, b):
    pad = jnp.tile(jnp.arange(_PADN, dtype=jnp.int32), _PADR)
    edges = jnp.concatenate(
        [edge_index, jnp.stack([pad, pad])], axis=1
    ).reshape(2, _ROWS, _LANE)
    zeros = jnp.zeros((_N, 1), jnp.float32)
    ones = jnp.ones((_LANE, 1), jnp.float32)

    deg0, deg1 = _deg_kernel(edges, zeros, ones)
    p, dis = _prep(deg0, deg1, x, W)
    acc0, acc1 = _agg_kernel(edges, p)
    out = _fin(acc0, acc1, p, dis, b.reshape(1, _OUT))
    return out


# speedup vs baseline: 84.2490x; 1.1056x over previous
import functools

import jax
import jax.numpy as jnp
from jax import lax
from jax.experimental import pallas as pl
from jax.experimental.pallas import tpu as pltpu
from jax.experimental.pallas import tpu_sc as plsc

_N = 100000
_E = 3200000
_OUT = 4
_LANE = 128
_NC = 2
_NS = 16
_ROWS = 25600
_RPW = _ROWS // (_NC * _NS)
_RC = 40
_NCHUNK = _RPW // _RC
_PADN = 7680
_PADR = 10
_DSL = 6248
_DSL_LAST = _N - (_NS - 1) * _DSL

_mesh = plsc.VectorSubcoreMesh(core_axis_name="core", subcore_axis_name="subcore")
_sc_params = pltpu.CompilerParams(use_tc_tiling_on_sc=False)


def _per_tile_slice(s, fn):
    @pl.when(s < _NS - 1)
    def _():
        fn(s * _DSL, _DSL)

    @pl.when(s == _NS - 1)
    def _():
        fn((_NS - 1) * _DSL, _DSL_LAST)


@functools.partial(
    pl.kernel,
    out_type=(
        jax.ShapeDtypeStruct((_N, 1), jnp.float32),
        jax.ShapeDtypeStruct((_N, 1), jnp.float32),
    ),
    mesh=_mesh,
    scratch_types=[
        pltpu.VMEM((_RC, _LANE), jnp.int32),
        pltpu.VMEM((_LANE, 1), jnp.float32),
        pltpu.VMEM_SHARED((_N, 1), jnp.float32),
        pltpu.SemaphoreType.DMA,
        pltpu.SemaphoreType.DMA,
    ],
    compiler_params=_sc_params,
)
def _deg_kernel(edges, zeros, ones, deg0, deg1, idxv, onesv, degsp, dsem0,
                dsem1):
    c = lax.axis_index("core")
    s = lax.axis_index("subcore")
    w = c * _NS + s

    pltpu.sync_copy(ones, onesv)

    def _zero(start, size):
        pltpu.sync_copy(zeros.at[pl.ds(start, size)],
                        degsp.at[pl.ds(start, size)])

    _per_tile_slice(s, _zero)
    plsc.subcore_barrier()

    @pl.loop(0, _NCHUNK)
    def _chunk(ci):
        row0 = w * _RPW + ci * _RC
        pltpu.sync_copy(edges.at[1, pl.ds(row0, _RC)], idxv)

        @pl.loop(0, _RC)
        def _row(j):
            pltpu.sync_copy(onesv, degsp.at[idxv.at[j]], add=True)

    plsc.subcore_barrier()

    def _read(start, size):
        @pl.when(c == 0)
        def _():
            pltpu.sync_copy(degsp.at[pl.ds(start, size)],
                            deg0.at[pl.ds(start, size)])

        @pl.when(c == 1)
        def _():
            pltpu.sync_copy(degsp.at[pl.ds(start, size)],
                            deg1.at[pl.ds(start, size)])

    _per_tile_slice(s, _read)


@functools.partial(
    pl.kernel,
    out_type=(
        jax.ShapeDtypeStruct((_N, _OUT), jnp.float32),
        jax.ShapeDtypeStruct((_N, _OUT), jnp.float32),
    ),
    mesh=_mesh,
    scratch_types=[
        pltpu.VMEM((_RC, _LANE), jnp.int32),
        pltpu.VMEM((_RC, _LANE), jnp.int32),
        pltpu.VMEM((_LANE, _OUT), jnp.float32),
        pltpu.VMEM((_LANE, _OUT), jnp.float32),
        pltpu.VMEM_SHARED((_N, _OUT), jnp.float32),
        pltpu.VMEM_SHARED((_N, _OUT), jnp.float32),
        pltpu.SemaphoreType.DMA,
        pltpu.SemaphoreType.DMA,
    ],
    compiler_params=_sc_params,
)
def _agg_kernel(edges, p, acc0, acc1, srcv, dstv, buf0, buf1, psp, accsp,
                gsem0, gsem1):
    c = lax.axis_index("core")
    s = lax.axis_index("subcore")
    w = c * _NS + s

    def _stage(start, size):
        pltpu.sync_copy(p.at[pl.ds(start, size)], psp.at[pl.ds(start, size)])
        pltpu.sync_copy(p.at[pl.ds(start, size)], accsp.at[pl.ds(start, size)])

    _per_tile_slice(s, _stage)
    plsc.subcore_barrier()

    @pl.loop(0, _NCHUNK)
    def _chunk(ci):
        row0 = w * _RPW + ci * _RC
        pltpu.sync_copy(edges.at[0, pl.ds(row0, _RC)], srcv)
        pltpu.sync_copy(edges.at[1, pl.ds(row0, _RC)], dstv)

        bufs = (buf0, buf1)
        sems = (gsem0, gsem1)
        pend = [None, None]
        pend[0] = pltpu.async_copy(psp.at[srcv.at[0]], buf0, gsem0)
        for j in range(_RC):
            b = j % 2
            if j + 1 < _RC:
                pend[1 - b] = pltpu.async_copy(
                    psp.at[srcv.at[j + 1]], bufs[1 - b], sems[1 - b]
                )
            pend[b].wait()
            pltpu.sync_copy(bufs[b], accsp.at[dstv.at[j]], add=True)

    plsc.subcore_barrier()

    def _read(start, size):
        @pl.when(c == 0)
        def _():
            pltpu.sync_copy(accsp.at[pl.ds(start, size)],
                            acc0.at[pl.ds(start, size)])

        @pl.when(c == 1)
        def _():
            pltpu.sync_copy(accsp.at[pl.ds(start, size)],
                            acc1.at[pl.ds(start, size)])

    _per_tile_slice(s, _read)


_BN = 5000


def _prep_body(d0_ref, d1_ref, x_ref, w_ref, p_ref, dis_ref):
    i = pl.program_id(0)
    gid = lax.broadcasted_iota(jnp.int32, (_BN, 1), 0) + i * _BN
    mask = (gid < _PADN).astype(jnp.float32)
    degt = d0_ref[...] + d1_ref[...] + 1.0 - float(_PADR) * mask
    dis = lax.rsqrt(degt)
    h = jnp.dot(x_ref[...], w_ref[...], preferred_element_type=jnp.float32)
    p_ref[...] = h * dis
    dis_ref[...] = dis


_prep = pl.pallas_call(
    _prep_body,
    grid=(_N // _BN,),
    in_specs=[
        pl.BlockSpec((_BN, 1), lambda i: (i, 0)),
        pl.BlockSpec((_BN, 1), lambda i: (i, 0)),
        pl.BlockSpec((_BN, 16), lambda i: (i, 0)),
        pl.BlockSpec((16, _OUT), lambda i: (0, 0)),
    ],
    out_specs=[
        pl.BlockSpec((_BN, _OUT), lambda i: (i, 0)),
        pl.BlockSpec((_BN, 1), lambda i: (i, 0)),
    ],
    out_shape=[
        jax.ShapeDtypeStruct((_N, _OUT), jnp.float32),
        jax.ShapeDtypeStruct((_N, 1), jnp.float32),
    ],
)


def _fin_body(a0_ref, a1_ref, p_ref, dis_ref, b_ref, o_ref):
    i = pl.program_id(0)
    gid = lax.broadcasted_iota(jnp.int32, (_BN, 1), 0) + i * _BN
    mask = (gid < _PADN).astype(jnp.float32)
    o_ref[...] = dis_ref[...] * (
        a0_ref[...] + a1_ref[...] - p_ref[...] * (1.0 + float(_PADR) * mask)
    ) + b_ref[...]


_fin = pl.pallas_call(
    _fin_body,
    grid=(_N // _BN,),
    in_specs=[
        pl.BlockSpec((_BN, _OUT), lambda i: (i, 0)),
        pl.BlockSpec((_BN, _OUT), lambda i: (i, 0)),
        pl.BlockSpec((_BN, _OUT), lambda i: (i, 0)),
        pl.BlockSpec((_BN, 1), lambda i: (i, 0)),
        pl.BlockSpec((1, _OUT), lambda i: (0, 0)),
    ],
    out_specs=pl.BlockSpec((_BN, _OUT), lambda i: (i, 0)),
    out_shape=jax.ShapeDtypeStruct((_N, _OUT), jnp.float32),
)


@jax.jit
def kernel(x, edge_index, W, b):
    pad = jnp.tile(jnp.arange(_PADN, dtype=jnp.int32), _PADR)
    edges = jnp.concatenate(
        [edge_index, jnp.stack([pad, pad])], axis=1
    ).reshape(2, _ROWS, _LANE)
    zeros = jnp.zeros((_N, 1), jnp.float32)
    ones = jnp.ones((_LANE, 1), jnp.float32)

    deg0, deg1 = _deg_kernel(edges, zeros, ones)
    p, dis = _prep(deg0, deg1, x, W)
    acc0, acc1 = _agg_kernel(edges, p)
    out = _fin(acc0, acc1, p, dis, b.reshape(1, _OUT))
    return out


# speedup vs baseline: 89.9060x; 1.0671x over previous
import functools

import jax
import jax.numpy as jnp
from jax import lax
from jax.experimental import pallas as pl
from jax.experimental.pallas import tpu as pltpu
from jax.experimental.pallas import tpu_sc as plsc

_N = 100000
_E = 3200000
_OUT = 4
_LANE = 128
_NC = 2
_NS = 16
_ROWS = 25600
_RPW = _ROWS // (_NC * _NS)
_RC = 40
_NCHUNK = _RPW // _RC
_PADN = 7680
_PADR = 10
_DSL = 6248
_DSL_LAST = _N - (_NS - 1) * _DSL

_mesh = plsc.VectorSubcoreMesh(core_axis_name="core", subcore_axis_name="subcore")
_sc_params = pltpu.CompilerParams(use_tc_tiling_on_sc=False)


def _per_tile_slice(s, fn):
    @pl.when(s < _NS - 1)
    def _():
        fn(s * _DSL, _DSL)

    @pl.when(s == _NS - 1)
    def _():
        fn((_NS - 1) * _DSL, _DSL_LAST)


@functools.partial(
    pl.kernel,
    out_type=(
        jax.ShapeDtypeStruct((_N, 1), jnp.float32),
        jax.ShapeDtypeStruct((_N, 1), jnp.float32),
    ),
    mesh=_mesh,
    scratch_types=[
        pltpu.VMEM((_RC, _LANE), jnp.int32),
        pltpu.VMEM((_LANE, 1), jnp.float32),
        pltpu.VMEM_SHARED((_N, 1), jnp.float32),
        [pltpu.SemaphoreType.DMA] * 4,
    ],
    compiler_params=_sc_params,
)
def _deg_kernel(edges, zeros, ones, deg0, deg1, idxv, onesv, degsp, dsems):
    c = lax.axis_index("core")
    s = lax.axis_index("subcore")
    w = c * _NS + s

    pltpu.sync_copy(ones, onesv)

    def _zero(start, size):
        pltpu.sync_copy(zeros.at[pl.ds(start, size)],
                        degsp.at[pl.ds(start, size)])

    _per_tile_slice(s, _zero)
    plsc.subcore_barrier()

    @pl.loop(0, _NCHUNK)
    def _chunk(ci):
        row0 = w * _RPW + ci * _RC
        pltpu.sync_copy(edges.at[1, pl.ds(row0, _RC)], idxv)

        pend = [None] * 4
        for j in range(_RC):
            b = j % 4
            if pend[b] is not None:
                pend[b].wait()
            pend[b] = pltpu.async_copy(onesv, degsp.at[idxv.at[j]],
                                       dsems[b], add=True)
        for b in range(4):
            pend[b].wait()

    plsc.subcore_barrier()

    def _read(start, size):
        @pl.when(c == 0)
        def _():
            pltpu.sync_copy(degsp.at[pl.ds(start, size)],
                            deg0.at[pl.ds(start, size)])

        @pl.when(c == 1)
        def _():
            pltpu.sync_copy(degsp.at[pl.ds(start, size)],
                            deg1.at[pl.ds(start, size)])

    _per_tile_slice(s, _read)


@functools.partial(
    pl.kernel,
    out_type=(
        jax.ShapeDtypeStruct((_N, _OUT), jnp.float32),
        jax.ShapeDtypeStruct((_N, _OUT), jnp.float32),
    ),
    mesh=_mesh,
    scratch_types=[
        pltpu.VMEM((_RC, _LANE), jnp.int32),
        pltpu.VMEM((_RC, _LANE), jnp.int32),
        [pltpu.VMEM((_LANE, _OUT), jnp.float32)] * 4,
        pltpu.VMEM_SHARED((_N, _OUT), jnp.float32),
        pltpu.VMEM_SHARED((_N, _OUT), jnp.float32),
        [pltpu.SemaphoreType.DMA] * 4,
        [pltpu.SemaphoreType.DMA] * 4,
    ],
    compiler_params=_sc_params,
)
def _agg_kernel(edges, p, acc0, acc1, srcv, dstv, bufs, psp, accsp,
                gsems, ssems):
    c = lax.axis_index("core")
    s = lax.axis_index("subcore")
    w = c * _NS + s

    def _stage(start, size):
        pltpu.sync_copy(p.at[pl.ds(start, size)], psp.at[pl.ds(start, size)])
        pltpu.sync_copy(p.at[pl.ds(start, size)], accsp.at[pl.ds(start, size)])

    _per_tile_slice(s, _stage)
    plsc.subcore_barrier()

    @pl.loop(0, _NCHUNK)
    def _chunk(ci):
        row0 = w * _RPW + ci * _RC
        pltpu.sync_copy(edges.at[0, pl.ds(row0, _RC)], srcv)
        pltpu.sync_copy(edges.at[1, pl.ds(row0, _RC)], dstv)

        nb = 4
        pend_g = [None] * nb
        pend_s = [None] * nb
        for j in range(nb - 1):
            pend_g[j] = pltpu.async_copy(psp.at[srcv.at[j]], bufs[j],
                                         gsems[j])
        for j in range(_RC):
            b = j % nb
            ahead = j + nb - 1
            if ahead < _RC:
                ab = ahead % nb
                if pend_s[ab] is not None:
                    pend_s[ab].wait()
                pend_g[ab] = pltpu.async_copy(psp.at[srcv.at[ahead]],
                                              bufs[ab], gsems[ab])
            pend_g[b].wait()
            pend_s[b] = pltpu.async_copy(bufs[b], accsp.at[dstv.at[j]],
                                         ssems[b], add=True)
        for b in range(nb):
            if pend_s[b] is not None:
                pend_s[b].wait()

    plsc.subcore_barrier()

    def _read(start, size):
        @pl.when(c == 0)
        def _():
            pltpu.sync_copy(accsp.at[pl.ds(start, size)],
                            acc0.at[pl.ds(start, size)])

        @pl.when(c == 1)
        def _():
            pltpu.sync_copy(accsp.at[pl.ds(start, size)],
                            acc1.at[pl.ds(start, size)])

    _per_tile_slice(s, _read)


_BN = 5000


def _mm_body(x_ref, w_ref, h_ref):
    h_ref[...] = jnp.dot(x_ref[...], w_ref[...],
                         preferred_element_type=jnp.float32)


_mm = pl.pallas_call(
    _mm_body,
    grid=(_N // _BN,),
    in_specs=[
        pl.BlockSpec((_BN, 16), lambda i: (i, 0)),
        pl.BlockSpec((16, _OUT), lambda i: (0, 0)),
    ],
    out_specs=pl.BlockSpec((_BN, _OUT), lambda i: (i, 0)),
    out_shape=jax.ShapeDtypeStruct((_N, _OUT), jnp.float32),
)


def _scale_body(d0_ref, d1_ref, h_ref, p_ref, dis_ref):
    i = pl.program_id(0)
    gid = lax.broadcasted_iota(jnp.int32, (_BN, 1), 0) + i * _BN
    mask = (gid < _PADN).astype(jnp.float32)
    degt = d0_ref[...] + d1_ref[...] + 1.0 - float(_PADR) * mask
    dis = lax.rsqrt(degt)
    p_ref[...] = h_ref[...] * dis
    dis_ref[...] = dis


_scale = pl.pallas_call(
    _scale_body,
    grid=(_N // _BN,),
    in_specs=[
        pl.BlockSpec((_BN, 1), lambda i: (i, 0)),
        pl.BlockSpec((_BN, 1), lambda i: (i, 0)),
        pl.BlockSpec((_BN, _OUT), lambda i: (i, 0)),
    ],
    out_specs=[
        pl.BlockSpec((_BN, _OUT), lambda i: (i, 0)),
        pl.BlockSpec((_BN, 1), lambda i: (i, 0)),
    ],
    out_shape=[
        jax.ShapeDtypeStruct((_N, _OUT), jnp.float32),
        jax.ShapeDtypeStruct((_N, 1), jnp.float32),
    ],
)


def _fin_body(a0_ref, a1_ref, p_ref, dis_ref, b_ref, o_ref):
    i = pl.program_id(0)
    gid = lax.broadcasted_iota(jnp.int32, (_BN, 1), 0) + i * _BN
    mask = (gid < _PADN).astype(jnp.float32)
    o_ref[...] = dis_ref[...] * (
        a0_ref[...] + a1_ref[...] - p_ref[...] * (1.0 + float(_PADR) * mask)
    ) + b_ref[...]


_fin = pl.pallas_call(
    _fin_body,
    grid=(_N // _BN,),
    in_specs=[
        pl.BlockSpec((_BN, _OUT), lambda i: (i, 0)),
        pl.BlockSpec((_BN, _OUT), lambda i: (i, 0)),
        pl.BlockSpec((_BN, _OUT), lambda i: (i, 0)),
        pl.BlockSpec((_BN, 1), lambda i: (i, 0)),
        pl.BlockSpec((1, _OUT), lambda i: (0, 0)),
    ],
    out_specs=pl.BlockSpec((_BN, _OUT), lambda i: (i, 0)),
    out_shape=jax.ShapeDtypeStruct((_N, _OUT), jnp.float32),
)


@jax.jit
def kernel(x, edge_index, W, b):
    pad = jnp.tile(jnp.arange(_PADN, dtype=jnp.int32), _PADR)
    edges = jnp.concatenate(
        [edge_index, jnp.stack([pad, pad])], axis=1
    ).reshape(2, _ROWS, _LANE)
    zeros = jnp.zeros((_N, 1), jnp.float32)
    ones = jnp.ones((_LANE, 1), jnp.float32)

    h = _mm(x, W)
    deg0, deg1 = _deg_kernel(edges, zeros, ones)
    p, dis = _scale(deg0, deg1, h)
    acc0, acc1 = _agg_kernel(edges, p)
    out = _fin(acc0, acc1, p, dis, b.reshape(1, _OUT))
    return out
